# baseline (speedup 1.0000x reference)
import functools

import jax
import jax.numpy as jnp
from jax import lax
from jax.experimental import pallas as pl
from jax.experimental.pallas import tpu as pltpu
from jax.experimental.pallas import tpu_sc as plsc

VOCAB = 1000
NC = 2
NS = 16
NW = NC * NS
CHUNK = 25


def _gather_body(b_per_w, L, idx_hbm, table_hbm, out_hbm, idx_v, table_sh,
                 buf0, buf1, sem0, sem1):
    c = lax.axis_index("c")
    s = lax.axis_index("s")
    wid = s * NC + c
    b_base = wid * b_per_w
    nchunk = b_per_w * L // CHUNK
    per_b = L // CHUNK
    bufs = (buf0, buf1)
    sems = (sem0, sem1)

    @pl.when(s < 8)
    def _():
        pltpu.sync_copy(table_hbm.at[pl.ds(s * 125, 125)],
                        table_sh.at[pl.ds(s * 125, 125)])

    pltpu.sync_copy(idx_hbm.at[wid], idx_v)
    plsc.subcore_barrier()

    pltpu.async_copy(table_sh.at[idx_v.at[0]], buf0, sem0)

    def step(i, carry):
        for p in range(2):
            cur = 2 * i + p
            nxt = cur + 1
            pltpu.make_async_copy(
                table_sh.at[idx_v.at[cur]], bufs[p], sems[p]).wait()
            @pl.when(nxt < nchunk)
            def _():
                pltpu.async_copy(
                    table_sh.at[idx_v.at[nxt]], bufs[1 - p], sems[1 - p])
            pltpu.sync_copy(
                bufs[p],
                out_hbm.at[b_base + cur // per_b,
                           pl.ds((cur % per_b) * CHUNK, CHUNK)])
        return carry

    lax.fori_loop(0, nchunk // 2, step, 0)


def kernel(idx, logits_table):
    B, L = idx.shape
    assert L % CHUNK == 0 and B % NW == 0
    b_per_w = B // NW
    nchunk = b_per_w * L // CHUNK
    assert nchunk % 2 == 0
    idx3 = idx.reshape(NW, nchunk, CHUNK).astype(jnp.int32)

    mesh = plsc.VectorSubcoreMesh(core_axis_name="c", subcore_axis_name="s")
    k = pl.kernel(
        functools.partial(_gather_body, b_per_w, L),
        out_type=jax.ShapeDtypeStruct((B, L, VOCAB), jnp.float32),
        mesh=mesh,
        scratch_types=[
            pltpu.VMEM((nchunk, CHUNK), jnp.int32),
            pltpu.VMEM_SHARED((VOCAB, VOCAB), jnp.float32),
            pltpu.VMEM((CHUNK, VOCAB), jnp.float32),
            pltpu.VMEM((CHUNK, VOCAB), jnp.float32),
            pltpu.SemaphoreType.DMA,
            pltpu.SemaphoreType.DMA,
        ],
        compiler_params=pltpu.CompilerParams(use_tc_tiling_on_sc=False),
    )
    return k(idx3, logits_table)


# speedup vs baseline: 1.1595x; 1.1595x over previous
import functools

import jax
import jax.numpy as jnp
from jax import lax
from jax.experimental import pallas as pl
from jax.experimental.pallas import tpu as pltpu
from jax.experimental.pallas import tpu_sc as plsc

VOCAB = 1000
VB = VOCAB // 8
NC = 2
NS = 16
NW = NC * NS
KMAX = -(-VB // NW)


def _body(B, L, idxT_hbm, tableT_hbm, out_hbm, tcols, irow0, irow1,
          obuf0, obuf1, isem0, isem1, osem0, osem1):
    c = lax.axis_index("c")
    s = lax.axis_index("s")
    w = s * NC + c
    nbb = B // 128
    irows = (irow0, irow1)
    isems = (isem0, isem1)
    obufs = (obuf0, obuf1)
    osems = (osem0, osem1)

    def build_block(irow, obuf):
        def g_body(g, carry):
            bb = g // 8
            bcoff = (g % 8) * 16
            iv = irow[pl.ds(g * 16, 16)]
            for vr in range(8):
                obuf[bb, vr, pl.ds(bcoff, 16)] = plsc.load_gather(
                    tcols.at[vr], [iv])
            return carry
        lax.fori_loop(0, (B // 16), g_body, 0)

    for kk in range(KMAX):
        vb = w + NW * kk

        @pl.when(vb < VB)
        def _():
            pltpu.sync_copy(tableT_hbm.at[pl.ds(vb * 8, 8)], tcols)
            pltpu.async_copy(idxT_hbm.at[0], irow0, isem0)

            def l_step(l, carry):
                for p in range(2):
                    li = 2 * l + p
                    pltpu.make_async_copy(
                        idxT_hbm.at[li], irows[p], isems[p]).wait()

                    @pl.when(li + 1 < L)
                    def _():
                        pltpu.async_copy(
                            idxT_hbm.at[li + 1], irows[1 - p], isems[1 - p])

                    @pl.when(li >= 2)
                    def _():
                        pltpu.make_async_copy(
                            obufs[p], out_hbm.at[li - 2, vb], osems[p]).wait()

                    build_block(irows[p], obufs[p])
                    pltpu.async_copy(
                        obufs[p], out_hbm.at[li, vb], osems[p])
                return carry

            lax.fori_loop(0, L // 2, l_step, 0)
            pltpu.make_async_copy(
                obufs[0], out_hbm.at[L - 2, vb], osems[0]).wait()
            pltpu.make_async_copy(
                obufs[1], out_hbm.at[L - 1, vb], osems[1]).wait()


def kernel(idx, logits_table):
    B, L = idx.shape
    assert B % (128 * NW) == 0 and L % 2 == 0 and VOCAB % 8 == 0
    idx_t = idx.T.astype(jnp.int32)
    table_t = logits_table.T

    mesh = plsc.VectorSubcoreMesh(core_axis_name="c", subcore_axis_name="s")
    k = pl.kernel(
        functools.partial(_body, B, L),
        out_type=jax.ShapeDtypeStruct((L, VB, B // 128, 8, 128), jnp.float32),
        mesh=mesh,
        scratch_types=[
            pltpu.VMEM((8, VOCAB), jnp.float32),
            pltpu.VMEM((B,), jnp.int32),
            pltpu.VMEM((B,), jnp.int32),
            pltpu.VMEM((B // 128, 8, 128), jnp.float32),
            pltpu.VMEM((B // 128, 8, 128), jnp.float32),
            pltpu.SemaphoreType.DMA,
            pltpu.SemaphoreType.DMA,
            pltpu.SemaphoreType.DMA,
            pltpu.SemaphoreType.DMA,
        ],
        compiler_params=pltpu.CompilerParams(use_tc_tiling_on_sc=False,
                                             needs_layout_passes=False),
    )
    out5 = k(idx_t, table_t)
    t = out5.transpose(2, 4, 0, 1, 3)
    return t.reshape(B, L, VOCAB)


# speedup vs baseline: 1.1643x; 1.0042x over previous
import functools

import jax
import jax.numpy as jnp
from jax import lax
from jax.experimental import pallas as pl
from jax.experimental.pallas import tpu as pltpu
from jax.experimental.pallas import tpu_sc as plsc

VOCAB = 1000
VB = VOCAB // 8
NC = 2
NS = 16
NW = NC * NS
KMAX = -(-VB // NW)


def _body(B, L, idxT_hbm, tableT_hbm, out_hbm, tcols, irow0, irow1,
          obuf0, obuf1, isem0, isem1, osem0, osem1):
    c = lax.axis_index("c")
    s = lax.axis_index("s")
    w = s * NC + c
    nbb = B // 128
    irows = (irow0, irow1)
    isems = (isem0, isem1)
    obufs = (obuf0, obuf1)
    osems = (osem0, osem1)

    def build_block(irow, obuf):
        def bb_body(bb, carry):
            base = bb * 128
            for gg in range(8):
                iv = irow[pl.ds(base + gg * 16, 16)]
                for vr in range(8):
                    obuf[bb, vr, pl.ds(gg * 16, 16)] = plsc.load_gather(
                        tcols.at[vr], [iv])
            return carry
        lax.fori_loop(0, nbb, bb_body, 0)

    for kk in range(KMAX):
        vb = w + NW * kk

        @pl.when(vb < VB)
        def _():
            pltpu.sync_copy(tableT_hbm.at[pl.ds(vb * 8, 8)], tcols)
            pltpu.async_copy(idxT_hbm.at[0], irow0, isem0)

            def l_step(l, carry):
                for p in range(2):
                    li = 2 * l + p
                    pltpu.make_async_copy(
                        idxT_hbm.at[li], irows[p], isems[p]).wait()

                    @pl.when(li + 1 < L)
                    def _():
                        pltpu.async_copy(
                            idxT_hbm.at[li + 1], irows[1 - p], isems[1 - p])

                    @pl.when(li >= 2)
                    def _():
                        pltpu.make_async_copy(
                            obufs[p], out_hbm.at[li - 2, vb], osems[p]).wait()

                    build_block(irows[p], obufs[p])
                    pltpu.async_copy(
                        obufs[p], out_hbm.at[li, vb], osems[p])
                return carry

            lax.fori_loop(0, L // 2, l_step, 0)
            pltpu.make_async_copy(
                obufs[0], out_hbm.at[L - 2, vb], osems[0]).wait()
            pltpu.make_async_copy(
                obufs[1], out_hbm.at[L - 1, vb], osems[1]).wait()


def kernel(idx, logits_table):
    B, L = idx.shape
    assert B % (128 * NW) == 0 and L % 2 == 0 and VOCAB % 8 == 0
    idx_t = idx.T.astype(jnp.int32)
    table_t = logits_table.T

    mesh = plsc.VectorSubcoreMesh(core_axis_name="c", subcore_axis_name="s")
    k = pl.kernel(
        functools.partial(_body, B, L),
        out_type=jax.ShapeDtypeStruct((L, VB, B // 128, 8, 128), jnp.float32),
        mesh=mesh,
        scratch_types=[
            pltpu.VMEM((8, VOCAB), jnp.float32),
            pltpu.VMEM((B,), jnp.int32),
            pltpu.VMEM((B,), jnp.int32),
            pltpu.VMEM((B // 128, 8, 128), jnp.float32),
            pltpu.VMEM((B // 128, 8, 128), jnp.float32),
            pltpu.SemaphoreType.DMA,
            pltpu.SemaphoreType.DMA,
            pltpu.SemaphoreType.DMA,
            pltpu.SemaphoreType.DMA,
        ],
        compiler_params=pltpu.CompilerParams(use_tc_tiling_on_sc=False,
                                             needs_layout_passes=False),
    )
    out5 = k(idx_t, table_t)
    t = out5.transpose(2, 4, 0, 1, 3)
    return t.reshape(B, L, VOCAB)


# speedup vs baseline: 5.4189x; 4.6541x over previous
import functools

import jax
import jax.numpy as jnp
from jax import lax
from jax.experimental import pallas as pl
from jax.experimental.pallas import tpu as pltpu
from jax.experimental.pallas import tpu_sc as plsc

VOCAB = 1000
VB = VOCAB // 8
NC = 2
NS = 16
NW = NC * NS
KMAX = -(-VB // NW)


def _body(B, L, idxT_hbm, tableT_hbm, out_hbm, tcols, irow0, irow1,
          obuf0, obuf1, isem0, isem1, osem0, osem1):
    c = lax.axis_index("c")
    s = lax.axis_index("s")
    w = s * NC + c
    nbb = B // 128
    irows = (irow0, irow1)
    isems = (isem0, isem1)
    obufs = (obuf0, obuf1)
    osems = (osem0, osem1)

    def build_block(irow, obuf):
        def bb_body(bb, carry):
            base = bb * 128
            ivs = [irow[pl.ds(base + gg * 16, 16)] for gg in range(8)]
            prev = None
            for gg in range(8):
                cur = []
                for vr in range(8):
                    cur.append(plsc.load_gather(tcols.at[vr], [ivs[gg]]))
                    if prev is not None:
                        obuf[bb, vr, pl.ds((gg - 1) * 16, 16)] = prev[vr]
                prev = cur
            for vr in range(8):
                obuf[bb, vr, pl.ds(7 * 16, 16)] = prev[vr]
            return carry
        lax.fori_loop(0, nbb, bb_body, 0)

    for kk in range(KMAX):
        vb = w + NW * kk

        @pl.when(vb < VB)
        def _():
            pltpu.sync_copy(tableT_hbm.at[pl.ds(vb * 8, 8)], tcols)
            pltpu.async_copy(idxT_hbm.at[0], irow0, isem0)

            def l_step(l, carry):
                for p in range(2):
                    li = 2 * l + p
                    pltpu.make_async_copy(
                        idxT_hbm.at[li], irows[p], isems[p]).wait()

                    @pl.when(li + 1 < L)
                    def _():
                        pltpu.async_copy(
                            idxT_hbm.at[li + 1], irows[1 - p], isems[1 - p])

                    @pl.when(li >= 2)
                    def _():
                        pltpu.make_async_copy(
                            obufs[p], out_hbm.at[li - 2, vb], osems[p]).wait()

                    build_block(irows[p], obufs[p])
                    pltpu.async_copy(
                        obufs[p], out_hbm.at[li, vb], osems[p])
                return carry

            lax.fori_loop(0, L // 2, l_step, 0)
            pltpu.make_async_copy(
                obufs[0], out_hbm.at[L - 2, vb], osems[0]).wait()
            pltpu.make_async_copy(
                obufs[1], out_hbm.at[L - 1, vb], osems[1]).wait()


def kernel(idx, logits_table):
    B, L = idx.shape
    assert B % (128 * NW) == 0 and L % 2 == 0 and VOCAB % 8 == 0
    idx_t = idx.T.astype(jnp.int32)
    table_t = logits_table.T

    mesh = plsc.VectorSubcoreMesh(core_axis_name="c", subcore_axis_name="s")
    k = pl.kernel(
        functools.partial(_body, B, L),
        out_type=jax.ShapeDtypeStruct((L, VB, B // 128, 8, 128), jnp.float32),
        mesh=mesh,
        scratch_types=[
            pltpu.VMEM((8, VOCAB), jnp.float32),
            pltpu.VMEM((B,), jnp.int32),
            pltpu.VMEM((B,), jnp.int32),
            pltpu.VMEM((B // 128, 8, 128), jnp.float32),
            pltpu.VMEM((B // 128, 8, 128), jnp.float32),
            pltpu.SemaphoreType.DMA,
            pltpu.SemaphoreType.DMA,
            pltpu.SemaphoreType.DMA,
            pltpu.SemaphoreType.DMA,
        ],
        compiler_params=pltpu.CompilerParams(use_tc_tiling_on_sc=False,
                                             needs_layout_passes=False),
    )
    out5 = k(idx_t, table_t)
    t = out5.transpose(2, 4, 0, 1, 3)
    return t.reshape(B, L, VOCAB)
